# baseline (speedup 1.0000x reference)
import jax
import jax.numpy as jnp
from jax.experimental import pallas as pl

_N_TOKENS = 4096
_IN = 2048
_NZ = 256
_NC = 8
_NCLS = 10
_PAD = 128
_BLK_M = 512
_SPLIT = 2
_SUB = _BLK_M // _SPLIT


def _fused(x_ref, w1, b1, w2, b2, w3, b3, wz, bz, ct, c2, *rest):
    ew = rest[:3 * _NC]
    eb = rest[3 * _NC:6 * _NC]
    out_ref = rest[6 * _NC]
    f32 = jnp.float32
    bf16 = jnp.bfloat16
    zero = jnp.bfloat16(0.0)

    hs = [x_ref[pl.ds(s * _SUB, _SUB), :].astype(bf16) for s in range(_SPLIT)]

    def layer(hs, w, b):
        return [jnp.maximum(jnp.dot(h, w[...], preferred_element_type=f32).astype(bf16) + b[...], zero)
                for h in hs]

    hs = layer(hs, w1, b1)
    hs = layer(hs, w2, b2)
    hs = layer(hs, w3, b3)
    zs = [jnp.dot(h, wz[...], preferred_element_type=f32) + bz[...] for h in hs]

    for s in range(_SPLIT):
        z = zs[s]
        z2 = jnp.sum(z * z, axis=1, keepdims=True)
        cross = jnp.dot(z, ct[...], preferred_element_type=f32)
        d2 = z2 - 2.0 * cross + c2[...]
        qu = 1.0 / (1.0 + d2)
        q = qu / jnp.sum(qu, axis=1, keepdims=True)

        zb = z.astype(bf16)
        acc = jnp.zeros((_SUB, _PAD), f32)
        for j in range(_NC):
            h1 = jnp.maximum(jnp.dot(zb, ew[3 * j][...], preferred_element_type=f32).astype(bf16) + eb[3 * j][...], zero)
            h2 = jnp.maximum(jnp.dot(h1, ew[3 * j + 1][...], preferred_element_type=f32).astype(bf16) + eb[3 * j + 1][...], zero)
            p = jnp.dot(h2, ew[3 * j + 2][...], preferred_element_type=f32) + eb[3 * j + 2][...]
            acc = acc + q[:, j:j + 1] * p
        out_ref[pl.ds(s * _SUB, _SUB), :] = acc


def kernel(X, enc_params, z_params, dec_params, xbar_params, cluster_layer, expert_params):
    bf16 = jnp.bfloat16
    f32 = jnp.float32

    (w1, b1), (w2, b2), (w3, b3) = enc_params
    wz, bz = z_params
    ews, ebs = [], []
    for j in range(_NC):
        (a1, c1), (a2, c2_), (a3, c3) = expert_params[j]
        ews.append(a1.astype(bf16))
        ews.append(a2.astype(bf16))
        ews.append(jnp.pad(a3, ((0, 0), (0, _PAD - _NCLS))).astype(bf16))
        ebs.append(c1.reshape(1, -1).astype(bf16))
        ebs.append(c2_.reshape(1, -1).astype(bf16))
        ebs.append(jnp.pad(c3, (0, _PAD - _NCLS)).reshape(1, _PAD))
    ct = cluster_layer.T.astype(f32)
    c2 = jnp.sum(cluster_layer * cluster_layer, axis=1).reshape(1, _NC)

    full = lambda shape: pl.BlockSpec(shape, lambda i: (0,) * len(shape))
    grid = (_N_TOKENS // _BLK_M,)
    ew_specs = [full((_NZ, 1024)), full((1024, 512)), full((512, _PAD))] * _NC
    eb_specs = [full((1, 1024)), full((1, 512)), full((1, _PAD))] * _NC
    out = pl.pallas_call(
        _fused,
        grid=grid,
        in_specs=[
            pl.BlockSpec((_BLK_M, _IN), lambda i: (i, 0)),
            full((_IN, _IN)), full((1, _IN)),
            full((_IN, _IN)), full((1, _IN)),
            full((_IN, 1024)), full((1, 1024)),
            full((1024, _NZ)), full((1, _NZ)),
            full((_NZ, _NC)), full((1, _NC)),
        ] + ew_specs + eb_specs,
        out_specs=pl.BlockSpec((_BLK_M, _PAD), lambda i: (i, 0)),
        out_shape=jax.ShapeDtypeStruct((_N_TOKENS, _PAD), f32),
    )(X,
      w1.astype(bf16), b1.reshape(1, -1).astype(bf16),
      w2.astype(bf16), b2.reshape(1, -1).astype(bf16),
      w3.astype(bf16), b3.reshape(1, -1).astype(bf16),
      wz.astype(bf16), bz.reshape(1, -1),
      ct, c2,
      *ews, *ebs)
    return out[:, :_NCLS]


# speedup vs baseline: 1.1166x; 1.1166x over previous
import jax
import jax.numpy as jnp
from jax.experimental import pallas as pl

_N_TOKENS = 4096
_IN = 2048
_NZ = 256
_NC = 8
_NCLS = 10
_PAD = 128
_BLK_M = 512


def _fused(x_ref, w1, b1, w2, b2, w3, b3, wz, bz, ct, c2,
           ew1, eb1, ew2, eb2, ew3, eb3, out_ref):
    f32 = jnp.float32
    bf16 = jnp.bfloat16
    x = x_ref[...]
    h = jnp.maximum(jnp.dot(x, w1[...], preferred_element_type=f32) + b1[...], 0.0).astype(bf16)
    h = jnp.maximum(jnp.dot(h, w2[...], preferred_element_type=f32) + b2[...], 0.0).astype(bf16)
    h = jnp.maximum(jnp.dot(h, w3[...], preferred_element_type=f32) + b3[...], 0.0).astype(bf16)
    z = jnp.dot(h, wz[...], preferred_element_type=f32) + bz[...]

    z2 = jnp.sum(z * z, axis=1, keepdims=True)
    cross = jnp.dot(z, ct[...], preferred_element_type=f32)
    d2 = z2 - 2.0 * cross + c2[...]
    qu = 1.0 / (1.0 + d2)
    q = qu / jnp.sum(qu, axis=1, keepdims=True)

    zb = z.astype(bf16)
    acc = jnp.zeros((_BLK_M, _PAD), f32)
    for j in range(_NC):
        h1 = jnp.maximum(jnp.dot(zb, ew1[j], preferred_element_type=f32) + eb1[j], 0.0).astype(bf16)
        h2 = jnp.maximum(jnp.dot(h1, ew2[j], preferred_element_type=f32) + eb2[j], 0.0).astype(bf16)
        p = jnp.dot(h2, ew3[j], preferred_element_type=f32) + eb3[j]
        acc = acc + q[:, j:j + 1] * p
    out_ref[...] = acc


def kernel(X, enc_params, z_params, dec_params, xbar_params, cluster_layer, expert_params):
    bf16 = jnp.bfloat16
    f32 = jnp.float32

    Xb = X.astype(bf16)
    (w1, b1), (w2, b2), (w3, b3) = enc_params
    wz, bz = z_params
    ew1 = jnp.stack([expert_params[j][0][0] for j in range(_NC)]).astype(bf16)
    eb1 = jnp.stack([expert_params[j][0][1] for j in range(_NC)]).reshape(_NC, 1, -1)
    ew2 = jnp.stack([expert_params[j][1][0] for j in range(_NC)]).astype(bf16)
    eb2 = jnp.stack([expert_params[j][1][1] for j in range(_NC)]).reshape(_NC, 1, -1)
    ew3 = jnp.pad(jnp.stack([expert_params[j][2][0] for j in range(_NC)]),
                  ((0, 0), (0, 0), (0, _PAD - _NCLS))).astype(bf16)
    eb3 = jnp.pad(jnp.stack([expert_params[j][2][1] for j in range(_NC)]),
                  ((0, 0), (0, _PAD - _NCLS))).reshape(_NC, 1, _PAD)
    ct = cluster_layer.T.astype(f32)
    c2 = jnp.sum(cluster_layer * cluster_layer, axis=1).reshape(1, _NC)

    full = lambda shape: pl.BlockSpec(shape, lambda i: (0,) * len(shape))
    grid = (_N_TOKENS // _BLK_M,)
    out = pl.pallas_call(
        _fused,
        grid=grid,
        in_specs=[
            pl.BlockSpec((_BLK_M, _IN), lambda i: (i, 0)),
            full((_IN, _IN)), full((1, _IN)),
            full((_IN, _IN)), full((1, _IN)),
            full((_IN, 1024)), full((1, 1024)),
            full((1024, _NZ)), full((1, _NZ)),
            full((_NZ, _NC)), full((1, _NC)),
            full((_NC, _NZ, 1024)), full((_NC, 1, 1024)),
            full((_NC, 1024, 512)), full((_NC, 1, 512)),
            full((_NC, 512, _PAD)), full((_NC, 1, _PAD)),
        ],
        out_specs=pl.BlockSpec((_BLK_M, _PAD), lambda i: (i, 0)),
        out_shape=jax.ShapeDtypeStruct((_N_TOKENS, _PAD), f32),
    )(Xb,
      w1.astype(bf16), b1.reshape(1, -1),
      w2.astype(bf16), b2.reshape(1, -1),
      w3.astype(bf16), b3.reshape(1, -1),
      wz.astype(bf16), bz.reshape(1, -1),
      ct, c2,
      ew1, eb1, ew2, eb2, ew3, eb3)
    return out[:, :_NCLS]


# speedup vs baseline: 1.2233x; 1.0956x over previous
import jax
import jax.numpy as jnp
from jax.experimental import pallas as pl

_N_TOKENS = 4096
_IN = 2048
_NZ = 256
_NC = 8
_NCLS = 10
_PAD = 128
_BLK_M = 512


def _fused(x_ref, w1, b1, w2, b2, w3, b3, wz, bz, ct, c2,
           ew1, eb1, ew2, eb2, ew3, eb3, out_ref):
    f32 = jnp.float32
    bf16 = jnp.bfloat16
    x = x_ref[...].astype(bf16)
    h = jnp.maximum(jnp.dot(x, w1[...], preferred_element_type=f32) + b1[...], 0.0).astype(bf16)
    h = jnp.maximum(jnp.dot(h, w2[...], preferred_element_type=f32) + b2[...], 0.0).astype(bf16)
    h = jnp.maximum(jnp.dot(h, w3[...], preferred_element_type=f32) + b3[...], 0.0).astype(bf16)
    z = jnp.dot(h, wz[...], preferred_element_type=f32) + bz[...]

    z2 = jnp.sum(z * z, axis=1, keepdims=True)
    cross = jnp.dot(z, ct[...], preferred_element_type=f32)
    d2 = z2 - 2.0 * cross + c2[...]
    qu = 1.0 / (1.0 + d2)
    q = qu / jnp.sum(qu, axis=1, keepdims=True)

    zb = z.astype(bf16)
    acc = jnp.zeros((_BLK_M, _PAD), f32)
    for j in range(_NC):
        h1 = jnp.maximum(jnp.dot(zb, ew1[j], preferred_element_type=f32) + eb1[j], 0.0).astype(bf16)
        h2 = jnp.maximum(jnp.dot(h1, ew2[j], preferred_element_type=f32) + eb2[j], 0.0).astype(bf16)
        p = jnp.dot(h2, ew3[j], preferred_element_type=f32) + eb3[j]
        acc = acc + q[:, j:j + 1] * p
    out_ref[...] = acc


def kernel(X, enc_params, z_params, dec_params, xbar_params, cluster_layer, expert_params):
    bf16 = jnp.bfloat16
    f32 = jnp.float32

    (w1, b1), (w2, b2), (w3, b3) = enc_params
    wz, bz = z_params
    ew1 = jnp.stack([expert_params[j][0][0] for j in range(_NC)]).astype(bf16)
    eb1 = jnp.stack([expert_params[j][0][1] for j in range(_NC)]).reshape(_NC, 1, -1)
    ew2 = jnp.stack([expert_params[j][1][0] for j in range(_NC)]).astype(bf16)
    eb2 = jnp.stack([expert_params[j][1][1] for j in range(_NC)]).reshape(_NC, 1, -1)
    ew3 = jnp.pad(jnp.stack([expert_params[j][2][0] for j in range(_NC)]),
                  ((0, 0), (0, 0), (0, _PAD - _NCLS))).astype(bf16)
    eb3 = jnp.pad(jnp.stack([expert_params[j][2][1] for j in range(_NC)]),
                  ((0, 0), (0, _PAD - _NCLS))).reshape(_NC, 1, _PAD)
    ct = cluster_layer.T.astype(f32)
    c2 = jnp.sum(cluster_layer * cluster_layer, axis=1).reshape(1, _NC)

    full = lambda shape: pl.BlockSpec(shape, lambda i: (0,) * len(shape))
    grid = (_N_TOKENS // _BLK_M,)
    out = pl.pallas_call(
        _fused,
        grid=grid,
        in_specs=[
            pl.BlockSpec((_BLK_M, _IN), lambda i: (i, 0)),
            full((_IN, _IN)), full((1, _IN)),
            full((_IN, _IN)), full((1, _IN)),
            full((_IN, 1024)), full((1, 1024)),
            full((1024, _NZ)), full((1, _NZ)),
            full((_NZ, _NC)), full((1, _NC)),
            full((_NC, _NZ, 1024)), full((_NC, 1, 1024)),
            full((_NC, 1024, 512)), full((_NC, 1, 512)),
            full((_NC, 512, _PAD)), full((_NC, 1, _PAD)),
        ],
        out_specs=pl.BlockSpec((_BLK_M, _PAD), lambda i: (i, 0)),
        out_shape=jax.ShapeDtypeStruct((_N_TOKENS, _PAD), f32),
    )(X,
      w1.astype(bf16), b1.reshape(1, -1),
      w2.astype(bf16), b2.reshape(1, -1),
      w3.astype(bf16), b3.reshape(1, -1),
      wz.astype(bf16), bz.reshape(1, -1),
      ct, c2,
      ew1, eb1, ew2, eb2, ew3, eb3)
    return out[:, :_NCLS]


# speedup vs baseline: 1.2246x; 1.0010x over previous
import jax
import jax.numpy as jnp
from jax.experimental import pallas as pl

_N_TOKENS = 4096
_IN = 2048
_NZ = 256
_NC = 8
_NCLS = 10
_PAD = 128
_BLK_M = 512


def _fused(x_ref, w1, b1, w2, b2, w3, b3, wz, bz, ct, c2,
           ew1, eb1, ew2, eb2, ew3, eb3, out_ref):
    f32 = jnp.float32
    bf16 = jnp.bfloat16
    x = x_ref[...].astype(bf16)
    h = jnp.maximum(jnp.dot(x, w1[...], preferred_element_type=f32) + b1[...], 0.0).astype(bf16)
    h = jnp.maximum(jnp.dot(h, w2[...], preferred_element_type=f32) + b2[...], 0.0).astype(bf16)
    h = jnp.maximum(jnp.dot(h, w3[...], preferred_element_type=f32) + b3[...], 0.0).astype(bf16)
    z = jnp.dot(h, wz[...], preferred_element_type=f32) + bz[...]

    z2 = jnp.sum(z * z, axis=1, keepdims=True)
    cross = jnp.dot(z, ct[...], preferred_element_type=f32)
    d2 = z2 - 2.0 * cross + c2[...]
    qu = 1.0 / (1.0 + d2)
    q = qu / jnp.sum(qu, axis=1, keepdims=True)

    zb = z.astype(bf16)
    acc = jnp.zeros((_BLK_M, _PAD), f32)
    for j in range(_NC):
        h1 = jnp.maximum(jnp.dot(zb, ew1[j], preferred_element_type=f32) + eb1[j], 0.0).astype(bf16)
        h2 = jnp.maximum(jnp.dot(h1, ew2[j], preferred_element_type=f32) + eb2[j], 0.0).astype(bf16)
        p = jnp.dot(h2, ew3[j], preferred_element_type=f32) + eb3[j]
        acc = acc + q[:, j:j + 1] * p
    out_ref[...] = acc[:, :_NCLS]


def kernel(X, enc_params, z_params, dec_params, xbar_params, cluster_layer, expert_params):
    bf16 = jnp.bfloat16
    f32 = jnp.float32

    (w1, b1), (w2, b2), (w3, b3) = enc_params
    wz, bz = z_params
    ew1 = jnp.stack([expert_params[j][0][0] for j in range(_NC)]).astype(bf16)
    eb1 = jnp.stack([expert_params[j][0][1] for j in range(_NC)]).reshape(_NC, 1, -1)
    ew2 = jnp.stack([expert_params[j][1][0] for j in range(_NC)]).astype(bf16)
    eb2 = jnp.stack([expert_params[j][1][1] for j in range(_NC)]).reshape(_NC, 1, -1)
    ew3 = jnp.pad(jnp.stack([expert_params[j][2][0] for j in range(_NC)]),
                  ((0, 0), (0, 0), (0, _PAD - _NCLS))).astype(bf16)
    eb3 = jnp.pad(jnp.stack([expert_params[j][2][1] for j in range(_NC)]),
                  ((0, 0), (0, _PAD - _NCLS))).reshape(_NC, 1, _PAD)
    ct = cluster_layer.T.astype(f32)
    c2 = jnp.sum(cluster_layer * cluster_layer, axis=1).reshape(1, _NC)

    full = lambda shape: pl.BlockSpec(shape, lambda i: (0,) * len(shape))
    grid = (_N_TOKENS // _BLK_M,)
    out = pl.pallas_call(
        _fused,
        grid=grid,
        in_specs=[
            pl.BlockSpec((_BLK_M, _IN), lambda i: (i, 0)),
            full((_IN, _IN)), full((1, _IN)),
            full((_IN, _IN)), full((1, _IN)),
            full((_IN, 1024)), full((1, 1024)),
            full((1024, _NZ)), full((1, _NZ)),
            full((_NZ, _NC)), full((1, _NC)),
            full((_NC, _NZ, 1024)), full((_NC, 1, 1024)),
            full((_NC, 1024, 512)), full((_NC, 1, 512)),
            full((_NC, 512, _PAD)), full((_NC, 1, _PAD)),
        ],
        out_specs=pl.BlockSpec((_BLK_M, _NCLS), lambda i: (i, 0)),
        out_shape=jax.ShapeDtypeStruct((_N_TOKENS, _NCLS), f32),
    )(X,
      w1.astype(bf16), b1.reshape(1, -1),
      w2.astype(bf16), b2.reshape(1, -1),
      w3.astype(bf16), b3.reshape(1, -1),
      wz.astype(bf16), bz.reshape(1, -1),
      ct, c2,
      ew1, eb1, ew2, eb2, ew3, eb3)
    return out


# speedup vs baseline: 1.2328x; 1.0067x over previous
import jax
import jax.numpy as jnp
from jax.experimental import pallas as pl

_N_TOKENS = 4096
_IN = 2048
_NZ = 256
_NC = 8
_NCLS = 10
_PAD = 128
_BLK_M = 512


def _fused(x_ref, w1, b1, w2, b2, w3, b3, wz, bz, ct, c2,
           ew1, eb1, ew2, eb2, ew3, eb3, out_ref):
    f32 = jnp.float32
    bf16 = jnp.bfloat16
    x = x_ref[...].astype(bf16)
    h = jnp.maximum(jnp.dot(x, w1[...], preferred_element_type=f32) + b1[...], 0.0).astype(bf16)
    h = jnp.maximum(jnp.dot(h, w2[...], preferred_element_type=f32) + b2[...], 0.0).astype(bf16)
    h = jnp.maximum(jnp.dot(h, w3[...], preferred_element_type=f32) + b3[...], 0.0).astype(bf16)
    z = jnp.dot(h, wz[...], preferred_element_type=f32) + bz[...]

    z2 = jnp.sum(z * z, axis=1, keepdims=True)
    cross = jnp.dot(z, ct[...], preferred_element_type=f32)
    d2 = z2 - 2.0 * cross + c2[...]
    qu = 1.0 / (1.0 + d2)
    q = qu / jnp.sum(qu, axis=1, keepdims=True)

    zb = z.astype(bf16)
    h1s = [jnp.maximum(jnp.dot(zb, ew1[:, c * 2048:(c + 1) * 2048], preferred_element_type=f32)
                       + eb1[:, c * 2048:(c + 1) * 2048], 0.0).astype(bf16)
           for c in range(4)]
    acc = jnp.zeros((_BLK_M, _PAD), f32)
    for j in range(_NC):
        h1 = h1s[j // 2][:, (j % 2) * 1024:(j % 2) * 1024 + 1024]
        h2 = jnp.maximum(jnp.dot(h1, ew2[j], preferred_element_type=f32) + eb2[j], 0.0).astype(bf16)
        p = jnp.dot(h2, ew3[j], preferred_element_type=f32) + eb3[j]
        acc = acc + q[:, j:j + 1] * p
    out_ref[...] = acc[:, :_NCLS]


def kernel(X, enc_params, z_params, dec_params, xbar_params, cluster_layer, expert_params):
    bf16 = jnp.bfloat16
    f32 = jnp.float32

    (w1, b1), (w2, b2), (w3, b3) = enc_params
    wz, bz = z_params
    ew1 = jnp.concatenate([expert_params[j][0][0] for j in range(_NC)], axis=1).astype(bf16)
    eb1 = jnp.concatenate([expert_params[j][0][1] for j in range(_NC)]).reshape(1, -1)
    ew2 = jnp.stack([expert_params[j][1][0] for j in range(_NC)]).astype(bf16)
    eb2 = jnp.stack([expert_params[j][1][1] for j in range(_NC)]).reshape(_NC, 1, -1)
    ew3 = jnp.pad(jnp.stack([expert_params[j][2][0] for j in range(_NC)]),
                  ((0, 0), (0, 0), (0, _PAD - _NCLS))).astype(bf16)
    eb3 = jnp.pad(jnp.stack([expert_params[j][2][1] for j in range(_NC)]),
                  ((0, 0), (0, _PAD - _NCLS))).reshape(_NC, 1, _PAD)
    ct = cluster_layer.T.astype(f32)
    c2 = jnp.sum(cluster_layer * cluster_layer, axis=1).reshape(1, _NC)

    full = lambda shape: pl.BlockSpec(shape, lambda i: (0,) * len(shape))
    grid = (_N_TOKENS // _BLK_M,)
    out = pl.pallas_call(
        _fused,
        grid=grid,
        in_specs=[
            pl.BlockSpec((_BLK_M, _IN), lambda i: (i, 0)),
            full((_IN, _IN)), full((1, _IN)),
            full((_IN, _IN)), full((1, _IN)),
            full((_IN, 1024)), full((1, 1024)),
            full((1024, _NZ)), full((1, _NZ)),
            full((_NZ, _NC)), full((1, _NC)),
            full((_NZ, _NC * 1024)), full((1, _NC * 1024)),
            full((_NC, 1024, 512)), full((_NC, 1, 512)),
            full((_NC, 512, _PAD)), full((_NC, 1, _PAD)),
        ],
        out_specs=pl.BlockSpec((_BLK_M, _NCLS), lambda i: (i, 0)),
        out_shape=jax.ShapeDtypeStruct((_N_TOKENS, _NCLS), f32),
    )(X,
      w1.astype(bf16), b1.reshape(1, -1),
      w2.astype(bf16), b2.reshape(1, -1),
      w3.astype(bf16), b3.reshape(1, -1),
      wz.astype(bf16), bz.reshape(1, -1),
      ct, c2,
      ew1, eb1, ew2, eb2, ew3, eb3)
    return out
